# baseline (speedup 1.0000x reference)
import numpy as np

import jax
import jax.numpy as jnp
from jax.experimental import pallas as pl
from jax.experimental.pallas import tpu as pltpu

_EMBED = [(33, 17), (2, 1), (8, 4), (3, 2), (3, 2), (3, 2), (3, 2), (3, 2), (20, 10)]
_VOFF = [0, 33, 35, 43, 46, 49, 52, 55, 58]
_DOFF = [0, 17, 18, 22, 24, 26, 28, 30, 32]
_B = 16384
_BLK = 2048
_NV = 80


def _body(x_ref, cont_ref, st_ref, tpad_ref, wpad_ref, wc_ref, bias_ref,
          gamma_ref, beta_ref, out_ref, m_ref):
    @pl.when(pl.program_id(0) == 0)
    def _fold():
        m_ref[...] = jnp.dot(tpad_ref[...], wpad_ref[...],
                             preferred_element_type=jnp.float32
                             ).astype(jnp.bfloat16)

    c = cont_ref[...]
    mean = jnp.mean(c)
    var = jnp.mean(c * c) - mean * mean
    inv = jax.lax.rsqrt(var + 1e-5)

    xb = x_ref[...]
    cn = (xb[:, 0:1] - mean) * inv * gamma_ref[...] + beta_ref[...]

    ones = jnp.ones((_BLK, 1), jnp.float32)
    xa = jnp.concatenate([xb[:, 1:10], ones], axis=1)
    tgt = jnp.dot(xa, st_ref[...], preferred_element_type=jnp.float32)
    colf = jax.lax.broadcasted_iota(jnp.int32, (_BLK, _NV), 1).astype(jnp.float32)
    ft = jnp.where(tgt == colf, 1.0, 0.0).astype(jnp.bfloat16)

    z = jnp.dot(ft, m_ref[...], preferred_element_type=jnp.float32)
    z = z + jnp.dot(cn, wc_ref[...], preferred_element_type=jnp.float32)
    z = z + bias_ref[...]
    mx = jnp.max(z, axis=1, keepdims=True)
    e = jnp.exp(z - mx)
    out_ref[...] = e / jnp.sum(e, axis=1, keepdims=True)


def kernel(x, emb0, emb1, emb2, emb3, emb4, emb5, emb6, emb7, emb8, W, b,
           gamma, beta):
    tables = [emb0, emb1, emb2, emb3, emb4, emb5, emb6, emb7, emb8]
    tpad = jnp.zeros((_NV, 128), jnp.float32)
    for i, (v, d) in enumerate(_EMBED):
        tpad = tpad.at[_VOFF[i]:_VOFF[i] + v, _DOFF[i]:_DOFF[i] + d].set(tables[i])
    wpad = jnp.zeros((128, 256), jnp.float32).at[:42, :].set(W[:, :42].T)

    st_np = np.zeros((10, _NV), np.float32)
    for i, (v, _) in enumerate(_EMBED):
        st_np[i, _VOFF[i]:_VOFF[i] + v] = 1.0
        st_np[9, _VOFF[i]:_VOFF[i] + v] = _VOFF[i]
    st_np[9, 78:] = -1.0
    st = jnp.asarray(st_np)

    cont_full = x[:, 0].reshape(8, 2048)
    grid = _B // _BLK

    out = pl.pallas_call(
        _body,
        grid=(grid,),
        in_specs=[
            pl.BlockSpec((_BLK, 10), lambda j: (j, 0)),
            pl.BlockSpec((8, 2048), lambda j: (0, 0)),
            pl.BlockSpec((10, _NV), lambda j: (0, 0)),
            pl.BlockSpec((_NV, 128), lambda j: (0, 0)),
            pl.BlockSpec((128, 256), lambda j: (0, 0)),
            pl.BlockSpec((1, 256), lambda j: (0, 0)),
            pl.BlockSpec((1, 256), lambda j: (0, 0)),
            pl.BlockSpec((1, 1), lambda j: (0, 0)),
            pl.BlockSpec((1, 1), lambda j: (0, 0)),
        ],
        out_specs=pl.BlockSpec((_BLK, 256), lambda j: (j, 0)),
        out_shape=jax.ShapeDtypeStruct((_B, 256), jnp.float32),
        scratch_shapes=[pltpu.VMEM((_NV, 256), jnp.bfloat16)],
    )(x, cont_full, st, tpad, wpad, W[:, 42].reshape(1, 256),
      b.reshape(1, 256), gamma.reshape(1, 1), beta.reshape(1, 1))
    return out


# speedup vs baseline: 1.2659x; 1.2659x over previous
import numpy as np

import jax
import jax.numpy as jnp
from jax.experimental import pallas as pl
from jax.experimental.pallas import tpu as pltpu

_EMBED = [(33, 17), (2, 1), (8, 4), (3, 2), (3, 2), (3, 2), (3, 2), (3, 2), (20, 10)]
_VOFF = [0, 33, 35, 43, 46, 49, 52, 55, 58]
_DOFF = [0, 17, 18, 22, 24, 26, 28, 30, 32]
_B = 16384
_BLK = 4096
_NV = 80


def _body(xt_ref, cont_ref, s_ref, tpad_ref, wpad_ref, wc_ref, bias_ref,
          gamma_ref, beta_ref, out_ref, m_ref):
    @pl.when(pl.program_id(0) == 0)
    def _fold():
        m_ref[...] = jnp.dot(tpad_ref[...], wpad_ref[...],
                             preferred_element_type=jnp.float32
                             ).astype(jnp.bfloat16)

    c = cont_ref[...]
    mean = jnp.mean(c)
    var = jnp.mean(c * c) - mean * mean
    inv = jax.lax.rsqrt(var + 1e-5)

    xt = xt_ref[...]
    cn = (xt[0:1, :] - mean) * inv * gamma_ref[...] + beta_ref[...]

    xa = jnp.concatenate([xt[1:10, :], jnp.ones((1, _BLK), jnp.float32)], 0)
    tgt = jnp.dot(s_ref[...], xa, preferred_element_type=jnp.float32)
    rowf = jax.lax.broadcasted_iota(jnp.int32, (_NV, _BLK), 0).astype(jnp.float32)
    ft = jnp.where(tgt == rowf, 1.0, 0.0).astype(jnp.bfloat16)

    z = jax.lax.dot_general(ft, m_ref[...],
                            dimension_numbers=(((0,), (0,)), ((), ())),
                            preferred_element_type=jnp.float32)
    z = z + jax.lax.dot_general(cn, wc_ref[...],
                                dimension_numbers=(((0,), (0,)), ((), ())),
                                preferred_element_type=jnp.float32)
    z = z + bias_ref[...]
    mx = jnp.max(z, axis=1, keepdims=True)
    e = jnp.exp(z - mx)
    out_ref[...] = e / jnp.sum(e, axis=1, keepdims=True)


def kernel(x, emb0, emb1, emb2, emb3, emb4, emb5, emb6, emb7, emb8, W, b,
           gamma, beta):
    tables = [emb0, emb1, emb2, emb3, emb4, emb5, emb6, emb7, emb8]
    tpad = jnp.zeros((_NV, 128), jnp.float32)
    for i, (v, d) in enumerate(_EMBED):
        tpad = tpad.at[_VOFF[i]:_VOFF[i] + v, _DOFF[i]:_DOFF[i] + d].set(tables[i])
    wpad = jnp.zeros((128, 256), jnp.float32).at[:42, :].set(W[:, :42].T)

    s_np = np.zeros((_NV, 10), np.float32)
    for i, (v, _) in enumerate(_EMBED):
        s_np[_VOFF[i]:_VOFF[i] + v, i] = 1.0
        s_np[_VOFF[i]:_VOFF[i] + v, 9] = _VOFF[i]
    s_np[78:, 9] = -1.0
    s = jnp.asarray(s_np)

    xt = x.T
    cont_full = x[:, 0].reshape(8, 2048)
    grid = _B // _BLK

    out = pl.pallas_call(
        _body,
        grid=(grid,),
        in_specs=[
            pl.BlockSpec((10, _BLK), lambda j: (0, j)),
            pl.BlockSpec((8, 2048), lambda j: (0, 0)),
            pl.BlockSpec((_NV, 10), lambda j: (0, 0)),
            pl.BlockSpec((_NV, 128), lambda j: (0, 0)),
            pl.BlockSpec((128, 256), lambda j: (0, 0)),
            pl.BlockSpec((1, 256), lambda j: (0, 0)),
            pl.BlockSpec((1, 256), lambda j: (0, 0)),
            pl.BlockSpec((1, 1), lambda j: (0, 0)),
            pl.BlockSpec((1, 1), lambda j: (0, 0)),
        ],
        out_specs=pl.BlockSpec((_BLK, 256), lambda j: (j, 0)),
        out_shape=jax.ShapeDtypeStruct((_B, 256), jnp.float32),
        scratch_shapes=[pltpu.VMEM((_NV, 256), jnp.bfloat16)],
    )(xt, cont_full, s, tpad, wpad, W[:, 42].reshape(1, 256),
      b.reshape(1, 256), gamma.reshape(1, 1), beta.reshape(1, 1))
    return out
